# baseline (speedup 1.0000x reference)
import functools

import jax
import jax.numpy as jnp
from jax import lax
from jax.experimental import pallas as pl
from jax.experimental.pallas import tpu as pltpu
from jax.experimental.pallas import tpu_sc as plsc

_N_FIELDS = 26
_VOCAB = 100000
_EMB_DIM = 32
_BATCH = 16384
_N_NUM = 13
_BN_EPS = 1e-5

_NC = 2
_NS = 16
_NW = _NC * _NS

_OUT_D = _N_FIELDS * _EMB_DIM + _N_NUM

_STRIP_B = 1024
_SPF = _BATCH // _STRIP_B
_N_STRIPS = _N_FIELDS * _SPF
_SPW = _N_STRIPS // _NW
_CHUNK = 128
_CPS = _STRIP_B // _CHUNK
_LANES = 128
_ROWS_PER_F = _STRIP_B * _EMB_DIM // _LANES


_HALF_F = _N_FIELDS // 2


def _sc_gather(tbl_lo, tbl_hi, idx):
    mesh = plsc.VectorSubcoreMesh(
        core_axis_name="c", subcore_axis_name="s",
        num_cores=_NC, num_subcores=_NS)

    @functools.partial(
        pl.kernel,
        out_type=jax.ShapeDtypeStruct((_N_STRIPS, _STRIP_B, _EMB_DIM),
                                      jnp.float32),
        mesh=mesh,
        scratch_types=[
            pltpu.VMEM((_SPW, _CPS, _CHUNK), jnp.int32),
            pltpu.VMEM((_STRIP_B, _EMB_DIM), jnp.float32),
            pltpu.VMEM((_STRIP_B, _EMB_DIM), jnp.float32),
            pltpu.SemaphoreType.DMA,
            pltpu.SemaphoreType.DMA,
        ],
        compiler_params=pltpu.CompilerParams(use_tc_tiling_on_sc=False),
    )
    def k(lo_hbm, hi_hbm, idx_hbm, out_hbm, idx_v, buf0, buf1, sem0, sem1):
        core = lax.axis_index("c")
        base_s = lax.axis_index("s") * _N_FIELDS + core * _HALF_F
        pltpu.sync_copy(idx_hbm.at[pl.ds(base_s, _SPW)], idx_v)

        bufs = (buf0, buf1)
        sems = (sem0, sem1)

        def run_half(tbl_hbm):
            def fire(sl, p):
                for j in range(_CPS):
                    pltpu.async_copy(
                        tbl_hbm.at[idx_v.at[sl, j]],
                        bufs[p].at[pl.ds(j * _CHUNK, _CHUNK)],
                        sems[p])

            def drain(sl, p):
                for j in range(_CPS):
                    pltpu.make_async_copy(
                        tbl_hbm.at[idx_v.at[sl, j]],
                        bufs[p].at[pl.ds(j * _CHUNK, _CHUNK)],
                        sems[p]).wait()

            def wout(sl, p):
                pltpu.sync_copy(bufs[p], out_hbm.at[base_s + sl])

            fire(0, 0)

            def strip_pair(h, carry):
                s0 = 2 * h
                fire(s0 + 1, 1)
                drain(s0, 0)
                wout(s0, 0)
                fire(s0 + 2, 0)
                drain(s0 + 1, 1)
                wout(s0 + 1, 1)
                return carry

            lax.fori_loop(0, (_SPW - 1) // 2, strip_pair, 0)
            drain(_SPW - 1, 0)
            wout(_SPW - 1, 0)

        @pl.when(core == 0)
        def _():
            run_half(lo_hbm)

        @pl.when(core == 1)
        def _():
            run_half(hi_hbm)

    return k(tbl_lo, tbl_hi, idx)


def _bn_body(xt_ref, g_ref, b_ref, o_ref):
    x = xt_ref[...]
    mean = jnp.mean(x, axis=1, keepdims=True)
    xc = x - mean
    var = jnp.mean(xc * xc, axis=1, keepdims=True)
    o_ref[...] = xc * lax.rsqrt(var + _BN_EPS) * g_ref[...] + b_ref[...]


def _asm_body(y_ref, cont_ref, o_ref):
    for f in range(_N_FIELDS):
        yf = y_ref[pl.ds(f * _ROWS_PER_F, _ROWS_PER_F), :]
        t = jnp.transpose(yf, (1, 0))
        for q in range(_STRIP_B // _ROWS_PER_F):
            o_ref[pl.ds(f * _EMB_DIM, _EMB_DIM),
                  pl.ds(q * _ROWS_PER_F, _ROWS_PER_F)] = (
                t[q * _EMB_DIM:(q + 1) * _EMB_DIM, :])
    o_ref[pl.ds(_N_FIELDS * _EMB_DIM, _N_NUM), :] = cont_ref[...]


def _assemble(strips_flat, cont_t):
    return pl.pallas_call(
        _asm_body,
        grid=(_SPF,),
        in_specs=[
            pl.BlockSpec((None, _N_FIELDS * _ROWS_PER_F, _LANES),
                         lambda b: (b, 0, 0)),
            pl.BlockSpec((_N_NUM, _STRIP_B), lambda b: (0, b)),
        ],
        out_specs=pl.BlockSpec((_OUT_D, _STRIP_B), lambda b: (0, b)),
        out_shape=jax.ShapeDtypeStruct((_OUT_D, _BATCH), jnp.float32),
    )(strips_flat, cont_t)


def kernel(x_numerical, x_cat, tables, gamma, beta):
    idx = (x_cat.T.astype(jnp.int32)
           + (jnp.arange(_N_FIELDS, dtype=jnp.int32) % _HALF_F)[:, None]
           * _VOCAB)
    idx = idx.reshape(_N_FIELDS, _SPF, 4, _ROWS_PER_F)
    idx = idx.transpose(1, 0, 3, 2)
    idx = idx.reshape(_N_STRIPS, _CPS, _CHUNK)
    tbl_lo = tables[:_HALF_F].reshape(_HALF_F * _VOCAB, _EMB_DIM)
    tbl_hi = tables[_HALF_F:].reshape(_HALF_F * _VOCAB, _EMB_DIM)

    cont_t = pl.pallas_call(
        _bn_body,
        out_shape=jax.ShapeDtypeStruct((_N_NUM, _BATCH), jnp.float32),
    )(x_numerical.T, gamma.reshape(_N_NUM, 1), beta.reshape(_N_NUM, 1))

    strips = _sc_gather(tbl_lo, tbl_hi, idx)
    strips_flat = strips.reshape(_SPF, _N_FIELDS * _ROWS_PER_F, _LANES)
    return _assemble(strips_flat, cont_t).T


# speedup vs baseline: 1.5374x; 1.5374x over previous
import functools

import jax
import jax.numpy as jnp
from jax import lax
from jax.experimental import pallas as pl
from jax.experimental.pallas import tpu as pltpu
from jax.experimental.pallas import tpu_sc as plsc

_N_FIELDS = 26
_VOCAB = 100000
_EMB_DIM = 32
_BATCH = 16384
_N_NUM = 13
_BN_EPS = 1e-5

_NC = 2
_NS = 16
_NW = _NC * _NS

_OUT_D = _N_FIELDS * _EMB_DIM + _N_NUM

_STRIP_B = 1024
_SPF = _BATCH // _STRIP_B
_N_STRIPS = _N_FIELDS * _SPF
_SPW = _N_STRIPS // _NW
_CHUNK = 128
_CPS = _STRIP_B // _CHUNK
_LANES = 128
_ROWS_PER_F = _STRIP_B * _EMB_DIM // _LANES


def _sc_gather(tables_flat, idx):
    mesh = plsc.VectorSubcoreMesh(
        core_axis_name="c", subcore_axis_name="s",
        num_cores=_NC, num_subcores=_NS)

    @functools.partial(
        pl.kernel,
        out_type=jax.ShapeDtypeStruct((_N_STRIPS, _STRIP_B, _EMB_DIM),
                                      jnp.float32),
        mesh=mesh,
        scratch_types=[
            pltpu.VMEM((_SPW, _CPS, _CHUNK), jnp.int32),
            pltpu.VMEM((_STRIP_B, _EMB_DIM), jnp.float32),
            pltpu.VMEM((_STRIP_B, _EMB_DIM), jnp.float32),
            pltpu.SemaphoreType.DMA,
            pltpu.SemaphoreType.DMA,
        ],
        compiler_params=pltpu.CompilerParams(use_tc_tiling_on_sc=False),
    )
    def k(tbl_hbm, idx_hbm, out_hbm, idx_v, buf0, buf1, sem0, sem1):
        wid = lax.axis_index("c") * _NS + lax.axis_index("s")
        pltpu.sync_copy(idx_hbm.at[pl.ds(wid * _SPW, _SPW)], idx_v)

        bufs = (buf0, buf1)
        sems = (sem0, sem1)

        def fire(sl, p):
            for j in range(_CPS):
                pltpu.async_copy(
                    tbl_hbm.at[idx_v.at[sl, j]],
                    bufs[p].at[pl.ds(j * _CHUNK, _CHUNK)],
                    sems[p])

        def drain(sl, p):
            for j in range(_CPS):
                pltpu.make_async_copy(
                    tbl_hbm.at[idx_v.at[sl, j]],
                    bufs[p].at[pl.ds(j * _CHUNK, _CHUNK)],
                    sems[p]).wait()

        def wout(sl, p):
            pltpu.sync_copy(bufs[p], out_hbm.at[wid * _SPW + sl])

        fire(0, 0)

        def strip_pair(h, carry):
            s0 = 2 * h
            fire(s0 + 1, 1)
            drain(s0, 0)
            wout(s0, 0)
            fire(s0 + 2, 0)
            drain(s0 + 1, 1)
            wout(s0 + 1, 1)
            return carry

        lax.fori_loop(0, (_SPW - 1) // 2, strip_pair, 0)
        drain(_SPW - 1, 0)
        wout(_SPW - 1, 0)

    return k(tables_flat, idx)


def _bn_body(xt_ref, g_ref, b_ref, o_ref):
    x = xt_ref[...]
    mean = jnp.mean(x, axis=1, keepdims=True)
    xc = x - mean
    var = jnp.mean(xc * xc, axis=1, keepdims=True)
    o_ref[...] = xc * lax.rsqrt(var + _BN_EPS) * g_ref[...] + b_ref[...]


def _asm_body(y_ref, cont_ref, o_ref):
    for f in range(_N_FIELDS):
        yf = y_ref[pl.ds(f * _ROWS_PER_F, _ROWS_PER_F), :]
        t = jnp.transpose(yf, (1, 0))
        for q in range(_STRIP_B // _ROWS_PER_F):
            o_ref[pl.ds(f * _EMB_DIM, _EMB_DIM),
                  pl.ds(q * _ROWS_PER_F, _ROWS_PER_F)] = (
                t[q * _EMB_DIM:(q + 1) * _EMB_DIM, :])
    o_ref[pl.ds(_N_FIELDS * _EMB_DIM, _N_NUM), :] = cont_ref[...]


def _assemble(strips_flat, cont_t):
    return pl.pallas_call(
        _asm_body,
        grid=(_SPF,),
        in_specs=[
            pl.BlockSpec((None, _N_FIELDS * _ROWS_PER_F, _LANES),
                         lambda b: (b, 0, 0)),
            pl.BlockSpec((_N_NUM, _STRIP_B), lambda b: (0, b)),
        ],
        out_specs=pl.BlockSpec((_OUT_D, _STRIP_B), lambda b: (0, b)),
        out_shape=jax.ShapeDtypeStruct((_OUT_D, _BATCH), jnp.float32),
    )(strips_flat, cont_t)


def kernel(x_numerical, x_cat, tables, gamma, beta):
    idx = (x_cat.T.astype(jnp.int32)
           + jnp.arange(_N_FIELDS, dtype=jnp.int32)[:, None] * _VOCAB)
    idx = idx.reshape(_N_FIELDS, _SPF, 4, _ROWS_PER_F)
    idx = idx.transpose(1, 0, 3, 2)
    idx = idx.reshape(_N_STRIPS, _CPS, _CHUNK)
    tables_flat = tables.reshape(_N_FIELDS * _VOCAB, _EMB_DIM)

    cont_t = pl.pallas_call(
        _bn_body,
        out_shape=jax.ShapeDtypeStruct((_N_NUM, _BATCH), jnp.float32),
    )(x_numerical.T, gamma.reshape(_N_NUM, 1), beta.reshape(_N_NUM, 1))

    strips = _sc_gather(tables_flat, idx)
    strips_flat = strips.reshape(_SPF, _N_FIELDS * _ROWS_PER_F, _LANES)
    return _assemble(strips_flat, cont_t).T
